# baseline (speedup 1.0000x reference)
import functools

import jax
import jax.numpy as jnp
from jax import lax
from jax.experimental import pallas as pl
from jax.experimental.pallas import tpu as pltpu
from jax.experimental.pallas import tpu_sc as plsc

_LANES = 16


def _pe_add_call(x, enc_weight, cls_row):
    B, T, D = x.shape
    T1 = T + 1
    dtype = x.dtype

    mesh = plsc.VectorSubcoreMesh(core_axis_name="c", subcore_axis_name="s")
    num_workers = mesh.num_cores * mesh.num_subcores
    assert T % num_workers == 0
    rows_per_worker = T // num_workers
    chunk = 16
    assert rows_per_worker % chunk == 0
    n_chunks = rows_per_worker // chunk
    n_vecs = D // _LANES
    nbuf = 3
    n_steps = n_chunks * B

    @functools.partial(
        pl.kernel,
        out_type=jax.ShapeDtypeStruct((B, T1, D), dtype),
        mesh=mesh,
        scratch_types=[
            pltpu.VMEM((chunk, D), dtype),
            pltpu.VMEM((chunk, D), dtype),
            pltpu.VMEM((chunk, D), dtype),
            pltpu.VMEM((chunk, D), dtype),
            pltpu.VMEM((chunk, D), dtype),
            pltpu.VMEM((B, 8, D), dtype),
            pltpu.VMEM((2, B, D), dtype),
            pltpu.VMEM((1, D), dtype),
            pltpu.VMEM((1, D), dtype),
            pltpu.VMEM((1, D), dtype),
            pltpu.SemaphoreType.DMA,
            pltpu.SemaphoreType.DMA,
            pltpu.SemaphoreType.DMA,
            pltpu.SemaphoreType.DMA,
            pltpu.SemaphoreType.DMA,
            pltpu.SemaphoreType.DMA,
            pltpu.SemaphoreType.DMA,
            pltpu.SemaphoreType.DMA,
            pltpu.SemaphoreType.DMA,
        ],
    )
    def pe_add(x_hbm, pe_hbm, cls_hbm, out_hbm,
               xb0, xb1, xb2, peb0, peb1, prevb, stash, cls_v, pe_last,
               row_tmp, sx0, sx1, sx2, so0, so1, so2, sp0, sp1, sprev):
        xb = [xb0, xb1, xb2]
        peb = [peb0, peb1]
        sx = [sx0, sx1, sx2]
        so = [so0, so1, so2]
        sp = [sp0, sp1]
        wid = lax.axis_index("s") * mesh.num_cores + lax.axis_index("c")
        base = wid * rows_per_worker

        def copy_row(dst, src):
            def jbody(j, carry):
                sl = pl.ds(j * _LANES, _LANES)
                dst[sl] = src[sl]
                return carry
            lax.fori_loop(0, n_vecs, jbody, 0)

        pe_d = [None] * n_chunks
        x_d = [None] * n_steps
        out_d = [None] * n_steps

        def start_x(s):
            c, b = s // B, s % B
            return pltpu.async_copy(
                x_hbm.at[b, pl.ds(base + c * chunk, chunk)], xb[s % nbuf],
                sx[s % nbuf])

        pe_d[0] = pltpu.async_copy(pe_hbm.at[pl.ds(base, chunk)], peb[0], sp[0])
        if n_chunks > 1:
            pe_d[1] = pltpu.async_copy(
                pe_hbm.at[pl.ds(base + chunk, chunk)], peb[1], sp[1])
        x_d[0] = start_x(0)

        @pl.when(wid != 0)
        def _():
            prev_d = [
                pltpu.async_copy(x_hbm.at[b, pl.ds(base - 8, 8)],
                                 prevb.at[b], sprev)
                for b in range(B)
            ]
            for b in range(B):
                prev_d[b].wait()
            for b in range(B):
                copy_row(stash.at[0, b], prevb.at[b, 7])

        @pl.when(wid == 0)
        def _():
            pltpu.sync_copy(cls_hbm, cls_v)
            for b in range(B):
                copy_row(stash.at[0, b], cls_v.at[0])

        for s in range(n_steps):
            c, b = s // B, s % B
            p = c % 2
            if s + 1 < n_steps:
                if s - (nbuf - 1) >= 0:
                    out_d[s - (nbuf - 1)].wait()
                x_d[s + 1] = start_x(s + 1)
            if b == 0:
                pe_d[c].wait()
                if 1 <= c and c + 1 < n_chunks:
                    pe_d[c + 1] = pltpu.async_copy(
                        pe_hbm.at[pl.ds(base + (c + 1) * chunk, chunk)],
                        peb[(c + 1) % 2], sp[(c + 1) % 2])
            x_d[s].wait()
            xv, pv = xb[s % nbuf], peb[p]

            copy_row(stash.at[1 - p, b], xv.at[chunk - 1])

            def row_add(i, carry):
                i2 = chunk - 1 - i
                for j in range(n_vecs):
                    sl = pl.ds(j * _LANES, _LANES)
                    xv[i2, sl] = xv[i2 - 1, sl] + pv[i2, sl]
                return carry

            lax.fori_loop(0, chunk - 1, row_add, 0)

            def row0(j, carry):
                sl = pl.ds(j * _LANES, _LANES)
                xv[0, sl] = stash[p, b, sl] + pv[0, sl]
                return carry

            lax.fori_loop(0, n_vecs, row0, 0)

            out_d[s] = pltpu.async_copy(
                xv, out_hbm.at[b, pl.ds(base + c * chunk, chunk)],
                so[s % nbuf])

        for s in range(max(0, n_steps - nbuf), n_steps):
            out_d[s].wait()

        @pl.when(wid == num_workers - 1)
        def _():
            pltpu.sync_copy(pe_hbm.at[pl.ds(T, 1)], pe_last)
            pfin = (n_chunks - 1) % 2
            for b in range(B):
                def fin(j, carry):
                    sl = pl.ds(j * _LANES, _LANES)
                    row_tmp[0, sl] = stash[1 - pfin, b, sl] + pe_last[0, sl]
                    return carry
                lax.fori_loop(0, n_vecs, fin, 0)
                pltpu.sync_copy(row_tmp, out_hbm.at[b, pl.ds(T, 1)])

    return pe_add(x, enc_weight, cls_row)


def kernel(x, enc_weight, cls_tokens_stream, cls_tokens_view, is_stream,
           stream_id, is_view, view_id, use_cls):
    B, T, D = x.shape
    cls_stream = lax.dynamic_slice_in_dim(cls_tokens_stream, stream_id, 1, axis=0)
    cls_view = lax.dynamic_slice_in_dim(cls_tokens_view, view_id, 1, axis=0)
    cls_zero = jnp.zeros((1, 1, D), dtype=x.dtype)
    cls_tok = jnp.where(
        jnp.asarray(is_stream) != 0,
        cls_stream,
        jnp.where(jnp.asarray(is_view) != 0, cls_view, cls_zero),
    )
    cls_tok = cls_tok * jnp.asarray(use_cls, dtype=x.dtype)
    cls_row = cls_tok.reshape(1, D)
    return _pe_add_call(x, enc_weight, cls_row)


# speedup vs baseline: 1.5195x; 1.5195x over previous
import functools

import jax
import jax.numpy as jnp
from jax import lax
from jax.experimental import pallas as pl
from jax.experimental.pallas import tpu as pltpu
from jax.experimental.pallas import tpu_sc as plsc

_LANES = 16


def _pe_add_call(x, enc_weight, cls_row):
    B, T, D = x.shape
    T1 = T + 1
    dtype = x.dtype

    mesh = plsc.VectorSubcoreMesh(core_axis_name="c", subcore_axis_name="s")
    num_workers = mesh.num_cores * mesh.num_subcores
    assert T % num_workers == 0
    rows_per_worker = T // num_workers
    chunk = 16
    assert rows_per_worker % chunk == 0
    n_chunks = rows_per_worker // chunk
    n_vecs = D // _LANES
    nbuf = 3
    n_steps = n_chunks * B

    @functools.partial(
        pl.kernel,
        out_type=jax.ShapeDtypeStruct((T1, B, D), dtype),
        mesh=mesh,
        scratch_types=[
            pltpu.VMEM((chunk, D), dtype),
            pltpu.VMEM((chunk, D), dtype),
            pltpu.VMEM((chunk, D), dtype),
            pltpu.VMEM((chunk, D), dtype),
            pltpu.VMEM((chunk, D), dtype),
            pltpu.VMEM((chunk, D), dtype),
            pltpu.VMEM((1, D), dtype),
            pltpu.VMEM((1, D), dtype),
            pltpu.SemaphoreType.DMA,
            pltpu.SemaphoreType.DMA,
            pltpu.SemaphoreType.DMA,
            pltpu.SemaphoreType.DMA,
            pltpu.SemaphoreType.DMA,
            pltpu.SemaphoreType.DMA,
            pltpu.SemaphoreType.DMA,
            pltpu.SemaphoreType.DMA,
            pltpu.SemaphoreType.DMA,
            pltpu.SemaphoreType.DMA,
        ],
    )
    def pe_add(x_hbm, pe_hbm, cls_hbm, out_hbm,
               xb0, xb1, xb2, peb0, peb1, peb2, pe_next, cls_v,
               sx0, sx1, sx2, so0, so1, so2, sp0, sp1, sp2, spn):
        xb = [xb0, xb1, xb2]
        peb = [peb0, peb1, peb2]
        sx = [sx0, sx1, sx2]
        so = [so0, so1, so2]
        sp = [sp0, sp1, sp2]
        wid = lax.axis_index("s") * mesh.num_cores + lax.axis_index("c")
        base = wid * rows_per_worker

        pe_d = [None] * n_chunks
        x_d = [None] * n_steps
        out_d = [None] * n_steps

        def start_x(s):
            c, b = s // B, s % B
            return pltpu.async_copy(
                x_hbm.at[b, pl.ds(base + c * chunk, chunk)], xb[s % nbuf],
                sx[s % nbuf])

        def start_pe(c):
            return pltpu.async_copy(
                pe_hbm.at[pl.ds(base + c * chunk, chunk)], peb[c % 3], sp[c % 3])

        pe_d[0] = start_pe(0)
        if n_chunks > 1:
            pe_d[1] = start_pe(1)
        pen_d = pltpu.async_copy(
            pe_hbm.at[pl.ds(base + rows_per_worker, 1)], pe_next, spn)
        x_d[0] = start_x(0)

        @pl.when(wid == 0)
        def _():
            pltpu.sync_copy(cls_hbm, cls_v)

        for s in range(n_steps):
            c, b = s // B, s % B
            if s + 1 < n_steps:
                if s - (nbuf - 1) >= 0:
                    out_d[s - (nbuf - 1)].wait()
                x_d[s + 1] = start_x(s + 1)
            if b == 0:
                if c == 0:
                    pe_d[0].wait()
                    if n_chunks > 1:
                        pe_d[1].wait()
                elif c + 1 < n_chunks:
                    pe_d[c + 1].wait()
                if c + 2 < n_chunks:
                    pe_d[c + 2] = start_pe(c + 2)
                if c == n_chunks - 1:
                    pen_d.wait()
            if s == 0:
                @pl.when(wid == 0)
                def _():
                    for j in range(n_vecs):
                        sl = pl.ds(j * _LANES, _LANES)
                        cls_v[0, sl] = cls_v[0, sl] + peb0[0, sl]
                    for b2 in range(B):
                        pltpu.sync_copy(cls_v, out_hbm.at[pl.ds(0, 1), b2])
            x_d[s].wait()
            xv, pv = xb[s % nbuf], peb[c % 3]
            pvn = peb[(c + 1) % 3] if c + 1 < n_chunks else pe_next

            def row_add(i, carry):
                for j in range(n_vecs):
                    sl = pl.ds(j * _LANES, _LANES)
                    xv[i, sl] = xv[i, sl] + pv[i + 1, sl]
                return carry

            lax.fori_loop(0, chunk - 1, row_add, 0)
            for j in range(n_vecs):
                sl = pl.ds(j * _LANES, _LANES)
                xv[chunk - 1, sl] = xv[chunk - 1, sl] + pvn[0, sl]

            out_d[s] = pltpu.async_copy(
                xv, out_hbm.at[pl.ds(base + c * chunk + 1, chunk), b],
                so[s % nbuf])

        for s in range(max(0, n_steps - nbuf), n_steps):
            out_d[s].wait()

    out_tbd = pe_add(x, enc_weight, cls_row)
    return jnp.transpose(out_tbd, (1, 0, 2))


def kernel(x, enc_weight, cls_tokens_stream, cls_tokens_view, is_stream,
           stream_id, is_view, view_id, use_cls):
    B, T, D = x.shape
    cls_stream = lax.dynamic_slice_in_dim(cls_tokens_stream, stream_id, 1, axis=0)
    cls_view = lax.dynamic_slice_in_dim(cls_tokens_view, view_id, 1, axis=0)
    cls_zero = jnp.zeros((1, 1, D), dtype=x.dtype)
    cls_tok = jnp.where(
        jnp.asarray(is_stream) != 0,
        cls_stream,
        jnp.where(jnp.asarray(is_view) != 0, cls_view, cls_zero),
    )
    cls_tok = cls_tok * jnp.asarray(use_cls, dtype=x.dtype)
    cls_row = cls_tok.reshape(1, D)
    return _pe_add_call(x, enc_weight, cls_row)
